# Initial kernel scaffold; baseline (speedup 1.0000x reference)
import jax
import jax.numpy as jnp
from jax.experimental import pallas as pl


def kernel(x, edge_index, W1_0, W1_1, b1, W2_0, W2_1, b2, W3_0, W3_1, b3):
    raise NotImplementedError("write your pallas kernel here")


# speedup vs baseline: 9.3057x; 9.3057x over previous
import functools

import jax
import jax.numpy as jnp
from jax import lax
from jax.experimental import pallas as pl
from jax.experimental.pallas import tpu as pltpu
from jax.experimental.pallas import tpu_sc as plsc

N = 10000
E = 320000
D = 128
NC = 2
NS = 16
NW = NC * NS
EPW = E // NW
CH = 80
NCHUNK = EPW // CH
NPAD = 10240
RPW = NPAD // NS
ZROWS = 128
ZRS = 16
BR = 1000


def _mesh():
    return plsc.VectorSubcoreMesh(
        core_axis_name="c", subcore_axis_name="s", num_cores=NC, num_subcores=NS
    )


def _sc_degree(dst3):
    @functools.partial(
        pl.kernel,
        out_type=jax.ShapeDtypeStruct((NC, NPAD, D), jnp.float32),
        mesh=_mesh(),
        scratch_types=[
            pltpu.VMEM((NCHUNK, CH), jnp.int32),
            pltpu.VMEM((CH, D), jnp.float32),
            pltpu.VMEM((ZRS, D), jnp.float32),
            pltpu.VMEM_SHARED((NPAD, D), jnp.float32),
        ],
    )
    def k(dst_hbm, out_hbm, idx_v, ones_v, zb_v, acc_s):
        cid = lax.axis_index("c")
        sid = lax.axis_index("s")
        wid = cid * NS + sid

        @pl.loop(0, CH)
        def _(i):
            for j in range(0, D, 16):
                ones_v[i, pl.ds(j, 16)] = jnp.ones((16,), jnp.float32)

        @pl.loop(0, ZRS)
        def _(i):
            for j in range(0, D, 16):
                zb_v[i, pl.ds(j, 16)] = jnp.zeros((16,), jnp.float32)

        pltpu.sync_copy(dst_hbm.at[wid], idx_v)

        @pl.loop(0, RPW, step=ZRS)
        def _(r):
            pltpu.sync_copy(zb_v, acc_s.at[pl.ds(sid * RPW + r, ZRS)])

        plsc.subcore_barrier()

        @pl.loop(0, NCHUNK)
        def _(i):
            pltpu.sync_copy(ones_v, acc_s.at[idx_v.at[i]], add=True)

        plsc.subcore_barrier()
        pltpu.sync_copy(
            acc_s.at[pl.ds(sid * RPW, RPW)],
            out_hbm.at[cid, pl.ds(sid * RPW, RPW)],
        )

    return k(dst3)


def _sc_scatter(q, src3, dst3):
    @functools.partial(
        pl.kernel,
        out_type=jax.ShapeDtypeStruct((NC, NPAD, D), jnp.float32),
        mesh=_mesh(),
        scratch_types=[
            pltpu.VMEM((NCHUNK, CH), jnp.int32),
            pltpu.VMEM((NCHUNK, CH), jnp.int32),
            pltpu.VMEM((CH, D), jnp.float32),
            pltpu.VMEM((ZRS, D), jnp.float32),
            pltpu.VMEM_SHARED((NPAD, D), jnp.float32),
            pltpu.SemaphoreType.DMA,
        ],
    )
    def k(q_hbm, src_hbm, dst_hbm, out_hbm, src_v, dst_v, rows_v, zb_v, acc_s, sem):
        cid = lax.axis_index("c")
        sid = lax.axis_index("s")
        wid = cid * NS + sid

        @pl.loop(0, ZRS)
        def _(i):
            for j in range(0, D, 16):
                zb_v[i, pl.ds(j, 16)] = jnp.zeros((16,), jnp.float32)

        pltpu.sync_copy(src_hbm.at[wid], src_v)
        pltpu.sync_copy(dst_hbm.at[wid], dst_v)

        @pl.loop(0, RPW, step=ZRS)
        def _(r):
            pltpu.sync_copy(zb_v, acc_s.at[pl.ds(sid * RPW + r, ZRS)])

        plsc.subcore_barrier()

        @pl.loop(0, NCHUNK)
        def _(i):
            pltpu.async_copy(q_hbm.at[src_v.at[i]], rows_v, sem).wait()
            pltpu.sync_copy(rows_v, acc_s.at[dst_v.at[i]], add=True)

        plsc.subcore_barrier()
        pltpu.sync_copy(
            acc_s.at[pl.ds(sid * RPW, RPW)],
            out_hbm.at[cid, pl.ds(sid * RPW, RPW)],
        )

    return k(q, src3, dst3)


def _first_body(x_ref, w0_ref, w1_ref, b_ref, deg_ref, t0_ref, q_ref, norm_ref):
    deg = deg_ref[0, :, 0:1] + deg_ref[1, :, 0:1]
    norm = lax.rsqrt(jnp.maximum(deg, 1.0))
    norm_ref[...] = norm
    xb = x_ref[...]
    t0_ref[...] = (
        jnp.dot(xb, w0_ref[...], preferred_element_type=jnp.float32)
        + b_ref[...][None, :]
    )
    q_ref[...] = jnp.dot(xb, w1_ref[...], preferred_element_type=jnp.float32) * norm


def _tc_first(x, w0, w1, b, deg):
    return pl.pallas_call(
        _first_body,
        grid=(N // BR,),
        in_specs=[
            pl.BlockSpec((BR, D), lambda i: (i, 0)),
            pl.BlockSpec((D, D), lambda i: (0, 0)),
            pl.BlockSpec((D, D), lambda i: (0, 0)),
            pl.BlockSpec((D,), lambda i: (0,)),
            pl.BlockSpec((NC, BR, D), lambda i: (0, i, 0)),
        ],
        out_specs=[
            pl.BlockSpec((BR, D), lambda i: (i, 0)),
            pl.BlockSpec((BR, D), lambda i: (i, 0)),
            pl.BlockSpec((BR, 1), lambda i: (i, 0)),
        ],
        out_shape=[
            jax.ShapeDtypeStruct((N, D), jnp.float32),
            jax.ShapeDtypeStruct((N, D), jnp.float32),
            jax.ShapeDtypeStruct((N, 1), jnp.float32),
        ],
    )(x, w0, w1, b, deg)


def _mid_body(t0p_ref, agg_ref, norm_ref, w0_ref, w1_ref, b_ref, t0_ref, q_ref):
    norm = norm_ref[...]
    h = jnp.tanh(t0p_ref[...] - (agg_ref[0] + agg_ref[1]) * norm)
    t0_ref[...] = (
        jnp.dot(h, w0_ref[...], preferred_element_type=jnp.float32)
        + b_ref[...][None, :]
    )
    q_ref[...] = jnp.dot(h, w1_ref[...], preferred_element_type=jnp.float32) * norm


def _tc_mid(t0p, agg, norm, w0, w1, b):
    return pl.pallas_call(
        _mid_body,
        grid=(N // BR,),
        in_specs=[
            pl.BlockSpec((BR, D), lambda i: (i, 0)),
            pl.BlockSpec((NC, BR, D), lambda i: (0, i, 0)),
            pl.BlockSpec((BR, 1), lambda i: (i, 0)),
            pl.BlockSpec((D, D), lambda i: (0, 0)),
            pl.BlockSpec((D, D), lambda i: (0, 0)),
            pl.BlockSpec((D,), lambda i: (0,)),
        ],
        out_specs=[
            pl.BlockSpec((BR, D), lambda i: (i, 0)),
            pl.BlockSpec((BR, D), lambda i: (i, 0)),
        ],
        out_shape=[
            jax.ShapeDtypeStruct((N, D), jnp.float32),
            jax.ShapeDtypeStruct((N, D), jnp.float32),
        ],
    )(t0p, agg, norm, w0, w1, b)


def _final_body(t0p_ref, agg_ref, norm_ref, out_ref):
    out_ref[...] = t0p_ref[...] - (agg_ref[0] + agg_ref[1]) * norm_ref[...]


def _tc_final(t0p, agg, norm):
    return pl.pallas_call(
        _final_body,
        grid=(N // BR,),
        in_specs=[
            pl.BlockSpec((BR, D), lambda i: (i, 0)),
            pl.BlockSpec((NC, BR, D), lambda i: (0, i, 0)),
            pl.BlockSpec((BR, 1), lambda i: (i, 0)),
        ],
        out_specs=pl.BlockSpec((BR, D), lambda i: (i, 0)),
        out_shape=jax.ShapeDtypeStruct((N, D), jnp.float32),
    )(t0p, agg, norm)


def kernel(x, edge_index, W1_0, W1_1, b1, W2_0, W2_1, b2, W3_0, W3_1, b3):
    src3 = edge_index[0].reshape(NW, NCHUNK, CH)
    dst3 = edge_index[1].reshape(NW, NCHUNK, CH)

    deg = _sc_degree(dst3)
    t0, q, norm = _tc_first(x, W1_0, W1_1, b1, deg)
    agg = _sc_scatter(q, src3, dst3)
    t0, q = _tc_mid(t0, agg, norm, W2_0, W2_1, b2)
    agg = _sc_scatter(q, src3, dst3)
    t0, q = _tc_mid(t0, agg, norm, W3_0, W3_1, b3)
    agg = _sc_scatter(q, src3, dst3)
    return _tc_final(t0, agg, norm)
